# baseline (speedup 1.0000x reference)
import jax
import jax.numpy as jnp
from jax import lax
from jax.experimental import pallas as pl
from jax.experimental.pallas import tpu as pltpu
from jax.experimental.pallas import tpu_sc as plsc

_NC = 2
_NS = 16
_NW = _NC * _NS
_L = 16

_C = 8
_NBUF = 3


def _sc_add_body(tokens_hbm, t16_hbm, emb_hbm, out_hbm,
                 emb_v, t_v, row_v,
                 in0, in1, in2, out0, out1, out2,
                 sem_in0, sem_in1, sem_in2, sem_out0, sem_out1, sem_out2):
    R, D = tokens_hbm.shape
    rows_per_w = R // _NW
    nchunks = rows_per_w // _C

    wid = lax.axis_index("s") * _NC + lax.axis_index("c")
    base = wid * rows_per_w

    pltpu.sync_copy(emb_hbm, emb_v)
    pltpu.sync_copy(t16_hbm, t_v)
    tvec = t_v[...]
    is_row0 = tvec == 0
    for j in range(D // _L):
        sl = pl.ds(j * _L, _L)
        row_v[sl] = jnp.where(is_row0, emb_v[0, sl], emb_v[1, sl])

    in_bufs = (in0, in1, in2)
    out_bufs = (out0, out1, out2)
    sems_in = (sem_in0, sem_in1, sem_in2)
    sems_out = (sem_out0, sem_out1, sem_out2)

    def start_in(c, b):
        pltpu.make_async_copy(
            tokens_hbm.at[pl.ds(base + c * _C, _C)], in_bufs[b], sems_in[b]
        ).start()

    def wait_in(b):
        pltpu.make_async_copy(
            tokens_hbm.at[pl.ds(base, _C)], in_bufs[b], sems_in[b]
        ).wait()

    def start_out(c, b):
        pltpu.make_async_copy(
            out_bufs[b], out_hbm.at[pl.ds(base + c * _C, _C)], sems_out[b]
        ).start()

    def wait_out(b):
        pltpu.make_async_copy(
            out_bufs[b], out_hbm.at[pl.ds(base, _C)], sems_out[b]
        ).wait()

    rounds = nchunks // _NBUF
    tail = nchunks - rounds * _NBUF

    def add_chunk(ib, ob):
        @plsc.parallel_loop(0, D // _L, unroll=8)
        def _(j):
            sl = pl.ds(j * _L, _L)
            rv = row_v[sl]
            for r in range(_C):
                ob[r, sl] = ib[r, sl] + rv

    for b in range(_NBUF):
        start_in(b, b)

    def round_body(g, _):
        for b in range(_NBUF):
            c = g * _NBUF + b
            wait_in(b)

            @pl.when(g >= 1)
            def _():
                wait_out(b)

            add_chunk(in_bufs[b], out_bufs[b])
            start_out(c, b)

            @pl.when(g + 1 < rounds)
            def _():
                start_in(c + _NBUF, b)
        return 0

    lax.fori_loop(0, rounds, round_body, 0)

    for i in range(tail):
        c = rounds * _NBUF + i
        wait_out(i)
        start_in(c, i)
        wait_in(i)
        add_chunk(in_bufs[i], out_bufs[i])
        start_out(c, i)

    for b in range(_NBUF):
        wait_out(b)


def kernel(tokens, t, emb):
    B, S, D = tokens.shape
    R = B * S
    flat = tokens.reshape(R, D)
    t16 = jnp.full((_L,), jnp.asarray(t, jnp.int32))

    mesh = plsc.VectorSubcoreMesh(core_axis_name="c", subcore_axis_name="s")
    run = pl.kernel(
        _sc_add_body,
        out_type=jax.ShapeDtypeStruct((R, D), tokens.dtype),
        mesh=mesh,
        scratch_types=[
            pltpu.VMEM((emb.shape[0], D), jnp.float32),
            pltpu.VMEM((_L,), jnp.int32),
            pltpu.VMEM((D,), jnp.float32),
            pltpu.VMEM((_C, D), jnp.float32),
            pltpu.VMEM((_C, D), jnp.float32),
            pltpu.VMEM((_C, D), jnp.float32),
            pltpu.VMEM((_C, D), jnp.float32),
            pltpu.VMEM((_C, D), jnp.float32),
            pltpu.VMEM((_C, D), jnp.float32),
            pltpu.SemaphoreType.DMA,
            pltpu.SemaphoreType.DMA,
            pltpu.SemaphoreType.DMA,
            pltpu.SemaphoreType.DMA,
            pltpu.SemaphoreType.DMA,
            pltpu.SemaphoreType.DMA,
        ],
    )
    out = run(flat, t16, emb)
    return out.reshape(B, S, D)


# speedup vs baseline: 1.1631x; 1.1631x over previous
import jax
import jax.numpy as jnp
from jax import lax
from jax.experimental import pallas as pl
from jax.experimental.pallas import tpu as pltpu
from jax.experimental.pallas import tpu_sc as plsc


def _sc_lookup_body(t1_hbm, emb_hbm, row_hbm, t_smem):
    cid = lax.axis_index("c")

    @pl.when(cid == 0)
    def _():
        pltpu.sync_copy(t1_hbm, t_smem)
        idx = t_smem[0]
        pltpu.sync_copy(emb_hbm.at[pl.ds(idx, 1)], row_hbm)


def _tc_add_body(x_ref, row_ref, o_ref):
    o_ref[...] = x_ref[...] + row_ref[...]


def kernel(tokens, t, emb):
    B, S, D = tokens.shape
    R = B * S
    flat = tokens.reshape(R, D)
    t1 = jnp.asarray(t, jnp.int32).reshape(1)

    mesh = plsc.ScalarSubcoreMesh(axis_name="c", num_cores=2)
    lookup = pl.kernel(
        _sc_lookup_body,
        out_type=jax.ShapeDtypeStruct((1, D), emb.dtype),
        mesh=mesh,
        scratch_types=[
            pltpu.SMEM((1,), jnp.int32),
        ],
    )
    row = lookup(t1, emb)

    BLK = 1024
    out = pl.pallas_call(
        _tc_add_body,
        grid=(R // BLK,),
        in_specs=[
            pl.BlockSpec((BLK, D), lambda i: (i, 0)),
            pl.BlockSpec((1, D), lambda i: (0, 0)),
        ],
        out_specs=pl.BlockSpec((BLK, D), lambda i: (i, 0)),
        out_shape=jax.ShapeDtypeStruct((R, D), tokens.dtype),
    )(flat, row)
    return out.reshape(B, S, D)


# speedup vs baseline: 1.1786x; 1.0133x over previous
import jax
import jax.numpy as jnp
from jax import lax
from jax.experimental import pallas as pl
from jax.experimental.pallas import tpu as pltpu
from jax.experimental.pallas import tpu_sc as plsc


def _sc_lookup_body(t1_hbm, emb_hbm, row_hbm, t_smem):
    cid = lax.axis_index("c")

    @pl.when(cid == 0)
    def _():
        pltpu.sync_copy(t1_hbm, t_smem)
        idx = t_smem[0]
        pltpu.sync_copy(emb_hbm.at[pl.ds(idx, 1)], row_hbm)


def _tc_add_body(x_ref, row_ref, o_ref):
    o_ref[...] = x_ref[...] + row_ref[...]


def kernel(tokens, t, emb):
    B, S, D = tokens.shape
    R = B * S
    flat = tokens.reshape(R, D)
    t1 = jnp.asarray(t, jnp.int32).reshape(1)

    mesh = plsc.ScalarSubcoreMesh(axis_name="c", num_cores=1)
    lookup = pl.kernel(
        _sc_lookup_body,
        out_type=jax.ShapeDtypeStruct((1, D), emb.dtype),
        mesh=mesh,
        scratch_types=[
            pltpu.SMEM((1,), jnp.int32),
        ],
    )
    row = lookup(t1, emb)

    BLK = 1024
    out = pl.pallas_call(
        _tc_add_body,
        grid=(R // BLK,),
        in_specs=[
            pl.BlockSpec((BLK, D), lambda i: (i, 0)),
            pl.BlockSpec((1, D), lambda i: (0, 0)),
        ],
        out_specs=pl.BlockSpec((BLK, D), lambda i: (i, 0)),
        out_shape=jax.ShapeDtypeStruct((R, D), tokens.dtype),
    )(flat, row)
    return out.reshape(B, S, D)
